# baseline (speedup 1.0000x reference)
import functools

import jax
import jax.numpy as jnp
from jax import lax
from jax.experimental import pallas as pl
from jax.experimental.pallas import tpu as pltpu
from jax.experimental.pallas import tpu_sc as plsc

NN = 10000
NP = 10240
EE = 320000
EP = 327680
EROWS = EP // 128
TROWS = EROWS // 16
DI = 128
DH = 256
DO = 128

_MESH = plsc.VectorSubcoreMesh(core_axis_name="c", subcore_axis_name="s")
_NPT = NP // 16


def _deg_body(dst2d, deg_out, ones_v, idx_v, deg_sh):
    c = lax.axis_index("c")
    s = lax.axis_index("s")

    @pl.when(c == 0)
    def _():
        @pl.loop(0, _NPT // 16)
        def _fill(i):
            ones_v[pl.ds(i * 16, 16)] = jnp.full((16,), 1.0, jnp.float32)

        pltpu.sync_copy(ones_v, deg_sh.at[pl.ds(s * _NPT, _NPT)])
        plsc.subcore_barrier()

        @pl.loop(0, TROWS // 16)
        def _chunk(j):
            base = s * TROWS + j * 16
            pltpu.sync_copy(dst2d.at[pl.ds(base, 16)], idx_v)

            @pl.loop(0, 16)
            def _row(r):
                pltpu.sync_copy(ones_v.at[pl.ds(0, 128)],
                                deg_sh.at[idx_v.at[r]], add=True)

        plsc.subcore_barrier()
        pltpu.sync_copy(deg_sh.at[pl.ds(s * _NPT, _NPT)],
                        deg_out.at[pl.ds(s * _NPT, _NPT)])


_deg_call = functools.partial(
    pl.kernel,
    out_type=jax.ShapeDtypeStruct((NP,), jnp.float32),
    mesh=_MESH,
    scratch_types=[
        pltpu.VMEM((_NPT,), jnp.float32),
        pltpu.VMEM((16, 128), jnp.int32),
        pltpu.VMEM_SHARED((NP,), jnp.float32),
    ],
)(_deg_body)


_C64 = 32
_NCH = (EP // 64) // 16 // _C64


def _prop_body(src2d, dst2d, tab_lo, tab_hi, out_lo, out_hi,
               srcb, dstb, b0, b1, b2, b3, acc_sh,
               g0, g1, g2, g3, s0, s1, s2, s3):
    c = lax.axis_index("c")
    s = lax.axis_index("s")
    bufs = (b0, b1, b2, b3)
    gsem = (g0, g1, g2, g3)
    ssem = (s0, s1, s2, s3)

    def run(table, out):
        pltpu.sync_copy(table.at[pl.ds(s * _NPT, _NPT)],
                        acc_sh.at[pl.ds(s * _NPT, _NPT)])
        plsc.subcore_barrier()

        @pl.loop(0, _NCH)
        def _chunk(j):
            base = (s * _NCH + j) * _C64
            pltpu.sync_copy(src2d.at[pl.ds(base, _C64)], srcb)
            pltpu.sync_copy(dst2d.at[pl.ds(base, _C64)], dstb)
            pltpu.async_copy(table.at[srcb.at[0]], bufs[0], gsem[0])
            pltpu.async_copy(table.at[srcb.at[1]], bufs[1], gsem[1])

            @pl.loop(0, _C64, step=4)
            def _row(r):
                for k in range(4):
                    i = r + k
                    kn = (k + 2) % 4
                    pltpu.make_async_copy(table.at[srcb.at[i]],
                                          bufs[k], gsem[k]).wait()
                    pltpu.async_copy(bufs[k], acc_sh.at[dstb.at[i]],
                                     ssem[k], add=True)

                    @pl.when(i >= 2)
                    def _():
                        pltpu.make_async_copy(
                            bufs[kn], acc_sh.at[dstb.at[i]], ssem[kn]).wait()

                    @pl.when(i + 2 < _C64)
                    def _():
                        pltpu.async_copy(table.at[srcb.at[i + 2]],
                                         bufs[kn], gsem[kn])

            pltpu.make_async_copy(bufs[2], acc_sh.at[dstb.at[0]],
                                  ssem[2]).wait()
            pltpu.make_async_copy(bufs[3], acc_sh.at[dstb.at[0]],
                                  ssem[3]).wait()

        plsc.subcore_barrier()
        pltpu.sync_copy(acc_sh.at[pl.ds(s * _NPT, _NPT)],
                        out.at[pl.ds(s * _NPT, _NPT)])

    @pl.when(c == 0)
    def _():
        run(tab_lo, out_lo)

    @pl.when(c == 1)
    def _():
        run(tab_hi, out_hi)


_prop_call = functools.partial(
    pl.kernel,
    out_type=[jax.ShapeDtypeStruct((NP, 128), jnp.float32),
              jax.ShapeDtypeStruct((NP, 128), jnp.float32)],
    mesh=_MESH,
    scratch_types=[
        pltpu.VMEM((_C64, 64), jnp.int32),
        pltpu.VMEM((_C64, 64), jnp.int32),
        pltpu.VMEM((64, 128), jnp.float32),
        pltpu.VMEM((64, 128), jnp.float32),
        pltpu.VMEM((64, 128), jnp.float32),
        pltpu.VMEM((64, 128), jnp.float32),
        pltpu.VMEM_SHARED((NP, 128), jnp.float32),
        pltpu.SemaphoreType.DMA,
        pltpu.SemaphoreType.DMA,
        pltpu.SemaphoreType.DMA,
        pltpu.SemaphoreType.DMA,
        pltpu.SemaphoreType.DMA,
        pltpu.SemaphoreType.DMA,
        pltpu.SemaphoreType.DMA,
        pltpu.SemaphoreType.DMA,
    ],
)(_prop_body)


def _prep_body(deg_ref, x_ref, w1_ref, zlo_ref, zhi_ref):
    dis = lax.rsqrt(deg_ref[...])
    z = jnp.dot(x_ref[...], w1_ref[...], preferred_element_type=jnp.float32)
    zs = z * dis
    zlo_ref[...] = zs[:, :128]
    zhi_ref[...] = zs[:, 128:]


def _mid_body(deg_ref, alo_ref, ahi_ref, wt_ref, wb_ref, blo_ref, bhi_ref,
              zlo_ref, zhi_ref):
    dis = lax.rsqrt(deg_ref[...])
    h_lo = jax.nn.relu(alo_ref[...] * dis + blo_ref[...])
    h_hi = jax.nn.relu(ahi_ref[...] * dis + bhi_ref[...])
    z2 = (jnp.dot(h_lo, wt_ref[...], preferred_element_type=jnp.float32)
          + jnp.dot(h_hi, wb_ref[...], preferred_element_type=jnp.float32))
    zs2 = z2 * dis
    zlo_ref[...] = zs2[:, :128]
    zhi_ref[...] = zs2[:, 128:]


def _final_body(deg_ref, alo_ref, ahi_ref, bmu_ref, bls_ref,
                mu_ref, ls_ref):
    dis = lax.rsqrt(deg_ref[...])
    mu_ref[...] = alo_ref[...] * dis + bmu_ref[...]
    ls_ref[...] = ahi_ref[...] * dis + bls_ref[...]


_BLK = 1024
_GRID = NP // _BLK

_row_spec = pl.BlockSpec((_BLK, 128), lambda i: (i, 0))
_deg_spec = pl.BlockSpec((_BLK, 1), lambda i: (i, 0))
_bias_spec = pl.BlockSpec((1, 128), lambda i: (0, 0))


def _prep_call(deg2, x_pad, w1):
    return pl.pallas_call(
        _prep_body,
        grid=(_GRID,),
        in_specs=[_deg_spec, _row_spec,
                  pl.BlockSpec((DI, DH), lambda i: (0, 0))],
        out_specs=[_row_spec, _row_spec],
        out_shape=[jax.ShapeDtypeStruct((NP, 128), jnp.float32)] * 2,
    )(deg2, x_pad, w1)


def _mid_call(deg2, alo, ahi, wt, wb, blo, bhi):
    return pl.pallas_call(
        _mid_body,
        grid=(_GRID,),
        in_specs=[_deg_spec, _row_spec, _row_spec,
                  pl.BlockSpec((128, DH), lambda i: (0, 0)),
                  pl.BlockSpec((128, DH), lambda i: (0, 0)),
                  _bias_spec, _bias_spec],
        out_specs=[_row_spec, _row_spec],
        out_shape=[jax.ShapeDtypeStruct((NP, 128), jnp.float32)] * 2,
    )(deg2, alo, ahi, wt, wb, blo, bhi)


def _final_call(deg2, alo, ahi, bmu, bls):
    return pl.pallas_call(
        _final_body,
        grid=(_GRID,),
        in_specs=[_deg_spec, _row_spec, _row_spec, _bias_spec, _bias_spec],
        out_specs=[_row_spec, _row_spec],
        out_shape=[jax.ShapeDtypeStruct((NP, 128), jnp.float32)] * 2,
    )(deg2, alo, ahi, bmu, bls)


def kernel(x, edge_index, W1, b1, W_mu, b_mu, W_ls, b_ls):
    src = edge_index[0]
    dst = edge_index[1]

    npad = EP - EE
    pad_src = (jnp.arange(npad, dtype=jnp.int32) * 61) % NN
    pad_dst = NN + (jnp.arange(npad, dtype=jnp.int32) % (NP - NN))
    src_all = jnp.concatenate([src, pad_src])
    dst_all = jnp.concatenate([dst, pad_dst])
    dst2d = dst_all.reshape(EROWS, 128)
    src2d64 = src_all.reshape(EP // 64, 64)
    dst2d64 = dst_all.reshape(EP // 64, 64)

    x_pad = jnp.pad(x, ((0, NP - NN), (0, 0)))

    wt = jnp.concatenate([W_mu[:128], W_ls[:128]], axis=1)
    wb = jnp.concatenate([W_mu[128:], W_ls[128:]], axis=1)
    blo = b1[:128].reshape(1, 128)
    bhi = b1[128:].reshape(1, 128)
    bmu = b_mu.reshape(1, 128)
    bls = b_ls.reshape(1, 128)

    deg = _deg_call(dst2d)
    deg2 = deg.reshape(NP, 1)

    zs_lo, zs_hi = _prep_call(deg2, x_pad, W1)
    acc_lo, acc_hi = _prop_call(src2d64, dst2d64, zs_lo, zs_hi)
    zs2_lo, zs2_hi = _mid_call(deg2, acc_lo, acc_hi, wt, wb, blo, bhi)
    acc2_lo, acc2_hi = _prop_call(src2d64, dst2d64, zs2_lo, zs2_hi)
    mu_p, ls_p = _final_call(deg2, acc2_lo, acc2_hi, bmu, bls)

    return (mu_p[:NN], ls_p[:NN])


# speedup vs baseline: 1.1317x; 1.1317x over previous
import functools

import jax
import jax.numpy as jnp
from jax import lax
from jax.experimental import pallas as pl
from jax.experimental.pallas import tpu as pltpu
from jax.experimental.pallas import tpu_sc as plsc

NN = 10000
NP = 10240
EE = 320000
EP = 327680
EROWS = EP // 128
TROWS = EROWS // 16
DI = 128
DH = 256
DO = 128

_MESH = plsc.VectorSubcoreMesh(core_axis_name="c", subcore_axis_name="s")
_NPT = NP // 16


def _deg_body(dst2d, deg_out, ones_v, idx_v, deg_sh):
    c = lax.axis_index("c")
    s = lax.axis_index("s")

    @pl.when(c == 0)
    def _():
        @pl.loop(0, _NPT // 16)
        def _fill(i):
            ones_v[pl.ds(i * 16, 16)] = jnp.full((16,), 1.0, jnp.float32)

        pltpu.sync_copy(ones_v, deg_sh.at[pl.ds(s * _NPT, _NPT)])
        plsc.subcore_barrier()

        @pl.loop(0, TROWS // 16)
        def _chunk(j):
            base = s * TROWS + j * 16
            pltpu.sync_copy(dst2d.at[pl.ds(base, 16)], idx_v)

            @pl.loop(0, 16)
            def _row(r):
                pltpu.sync_copy(ones_v.at[pl.ds(0, 128)],
                                deg_sh.at[idx_v.at[r]], add=True)

        plsc.subcore_barrier()
        pltpu.sync_copy(deg_sh.at[pl.ds(s * _NPT, _NPT)],
                        deg_out.at[pl.ds(s * _NPT, _NPT)])


_deg_call = functools.partial(
    pl.kernel,
    out_type=jax.ShapeDtypeStruct((NP,), jnp.float32),
    mesh=_MESH,
    scratch_types=[
        pltpu.VMEM((_NPT,), jnp.float32),
        pltpu.VMEM((16, 128), jnp.int32),
        pltpu.VMEM_SHARED((NP,), jnp.float32),
    ],
)(_deg_body)


_CHK = 32


def _prop_body(src2d, dst2d, tab_lo, tab_hi, out_lo, out_hi,
               srcb, dstb, buf0, buf1, acc_sh, sem0, sem1):
    c = lax.axis_index("c")
    s = lax.axis_index("s")

    def run(table, out):
        pltpu.sync_copy(table.at[pl.ds(s * _NPT, _NPT)],
                        acc_sh.at[pl.ds(s * _NPT, _NPT)])
        plsc.subcore_barrier()

        @pl.loop(0, TROWS // _CHK)
        def _chunk(j):
            base = s * TROWS + j * _CHK
            pltpu.sync_copy(src2d.at[pl.ds(base, _CHK)], srcb)
            pltpu.sync_copy(dst2d.at[pl.ds(base, _CHK)], dstb)
            pltpu.async_copy(table.at[srcb.at[0]], buf0, sem0)
            pltpu.async_copy(table.at[srcb.at[1]], buf1, sem1)

            @pl.loop(0, _CHK, step=2)
            def _row(r):
                for k, (buf, sem) in enumerate(((buf0, sem0), (buf1, sem1))):
                    idx = r + k
                    pltpu.make_async_copy(table.at[srcb.at[idx]],
                                          buf, sem).wait()
                    pltpu.sync_copy(buf, acc_sh.at[dstb.at[idx]], add=True)

                    @pl.when(idx + 2 < _CHK)
                    def _():
                        pltpu.async_copy(table.at[srcb.at[idx + 2]], buf, sem)

        plsc.subcore_barrier()
        pltpu.sync_copy(acc_sh.at[pl.ds(s * _NPT, _NPT)],
                        out.at[pl.ds(s * _NPT, _NPT)])

    @pl.when(c == 0)
    def _():
        run(tab_lo, out_lo)

    @pl.when(c == 1)
    def _():
        run(tab_hi, out_hi)


_prop_call = functools.partial(
    pl.kernel,
    out_type=[jax.ShapeDtypeStruct((NP, 128), jnp.float32),
              jax.ShapeDtypeStruct((NP, 128), jnp.float32)],
    mesh=_MESH,
    scratch_types=[
        pltpu.VMEM((_CHK, 128), jnp.int32),
        pltpu.VMEM((_CHK, 128), jnp.int32),
        pltpu.VMEM((128, 128), jnp.float32),
        pltpu.VMEM((128, 128), jnp.float32),
        pltpu.VMEM_SHARED((NP, 128), jnp.float32),
        pltpu.SemaphoreType.DMA,
        pltpu.SemaphoreType.DMA,
    ],
)(_prop_body)


def _prep_body(deg_ref, x_ref, w1_ref, zlo_ref, zhi_ref):
    dis = lax.rsqrt(deg_ref[...])
    z = jnp.dot(x_ref[...], w1_ref[...], preferred_element_type=jnp.float32)
    zs = z * dis
    zlo_ref[...] = zs[:, :128]
    zhi_ref[...] = zs[:, 128:]


def _mid_body(deg_ref, alo_ref, ahi_ref, wt_ref, wb_ref, blo_ref, bhi_ref,
              zlo_ref, zhi_ref):
    dis = lax.rsqrt(deg_ref[...])
    h_lo = jax.nn.relu(alo_ref[...] * dis + blo_ref[...])
    h_hi = jax.nn.relu(ahi_ref[...] * dis + bhi_ref[...])
    z2 = (jnp.dot(h_lo, wt_ref[...], preferred_element_type=jnp.float32)
          + jnp.dot(h_hi, wb_ref[...], preferred_element_type=jnp.float32))
    zs2 = z2 * dis
    zlo_ref[...] = zs2[:, :128]
    zhi_ref[...] = zs2[:, 128:]


def _final_body(deg_ref, alo_ref, ahi_ref, bmu_ref, bls_ref,
                mu_ref, ls_ref):
    dis = lax.rsqrt(deg_ref[...])
    mu_ref[...] = alo_ref[...] * dis + bmu_ref[...]
    ls_ref[...] = ahi_ref[...] * dis + bls_ref[...]


_BLK = 1024
_GRID = NP // _BLK

_row_spec = pl.BlockSpec((_BLK, 128), lambda i: (i, 0))
_deg_spec = pl.BlockSpec((_BLK, 1), lambda i: (i, 0))
_bias_spec = pl.BlockSpec((1, 128), lambda i: (0, 0))


def _prep_call(deg2, x_pad, w1):
    return pl.pallas_call(
        _prep_body,
        grid=(_GRID,),
        in_specs=[_deg_spec, _row_spec,
                  pl.BlockSpec((DI, DH), lambda i: (0, 0))],
        out_specs=[_row_spec, _row_spec],
        out_shape=[jax.ShapeDtypeStruct((NP, 128), jnp.float32)] * 2,
    )(deg2, x_pad, w1)


def _mid_call(deg2, alo, ahi, wt, wb, blo, bhi):
    return pl.pallas_call(
        _mid_body,
        grid=(_GRID,),
        in_specs=[_deg_spec, _row_spec, _row_spec,
                  pl.BlockSpec((128, DH), lambda i: (0, 0)),
                  pl.BlockSpec((128, DH), lambda i: (0, 0)),
                  _bias_spec, _bias_spec],
        out_specs=[_row_spec, _row_spec],
        out_shape=[jax.ShapeDtypeStruct((NP, 128), jnp.float32)] * 2,
    )(deg2, alo, ahi, wt, wb, blo, bhi)


def _final_call(deg2, alo, ahi, bmu, bls):
    return pl.pallas_call(
        _final_body,
        grid=(_GRID,),
        in_specs=[_deg_spec, _row_spec, _row_spec, _bias_spec, _bias_spec],
        out_specs=[_row_spec, _row_spec],
        out_shape=[jax.ShapeDtypeStruct((NP, 128), jnp.float32)] * 2,
    )(deg2, alo, ahi, bmu, bls)


def kernel(x, edge_index, W1, b1, W_mu, b_mu, W_ls, b_ls):
    src = edge_index[0]
    dst = edge_index[1]

    npad = EP - EE
    pad_src = (jnp.arange(npad, dtype=jnp.int32) * 61) % NN
    pad_dst = NN + (jnp.arange(npad, dtype=jnp.int32) % (NP - NN))
    src2d = jnp.concatenate([src, pad_src]).reshape(EROWS, 128)
    dst2d = jnp.concatenate([dst, pad_dst]).reshape(EROWS, 128)

    x_pad = jnp.pad(x, ((0, NP - NN), (0, 0)))

    wt = jnp.concatenate([W_mu[:128], W_ls[:128]], axis=1)
    wb = jnp.concatenate([W_mu[128:], W_ls[128:]], axis=1)
    blo = b1[:128].reshape(1, 128)
    bhi = b1[128:].reshape(1, 128)
    bmu = b_mu.reshape(1, 128)
    bls = b_ls.reshape(1, 128)

    deg = _deg_call(dst2d)
    deg2 = deg.reshape(NP, 1)

    zs_lo, zs_hi = _prep_call(deg2, x_pad, W1)
    acc_lo, acc_hi = _prop_call(src2d, dst2d, zs_lo, zs_hi)
    zs2_lo, zs2_hi = _mid_call(deg2, acc_lo, acc_hi, wt, wb, blo, bhi)
    acc2_lo, acc2_hi = _prop_call(src2d, dst2d, zs2_lo, zs2_hi)
    mu_p, ls_p = _final_call(deg2, acc2_lo, acc2_hi, bmu, bls)

    return (mu_p[:NN], ls_p[:NN])


# speedup vs baseline: 1.1655x; 1.0299x over previous
import functools

import jax
import jax.numpy as jnp
from jax import lax
from jax.experimental import pallas as pl
from jax.experimental.pallas import tpu as pltpu
from jax.experimental.pallas import tpu_sc as plsc

NN = 10000
NP = 10240
EE = 320000
EP = 327680
EROWS = EP // 128
TROWS = EROWS // 16
DI = 128
DH = 256
DO = 128

_MESH = plsc.VectorSubcoreMesh(core_axis_name="c", subcore_axis_name="s")
_NPT = NP // 16


_DROWS = EROWS // 2 // 16


def _deg_body(dst2d, d0_out, d1_out, half_v, ones_v, idx_v, deg_sh):
    c = lax.axis_index("c")
    s = lax.axis_index("s")

    @pl.loop(0, _NPT // 16)
    def _fill(i):
        half_v[pl.ds(i * 16, 16)] = jnp.full((16,), 0.5, jnp.float32)

    @pl.loop(0, 128 // 16)
    def _fill1(i):
        ones_v[pl.ds(i * 16, 16)] = jnp.full((16,), 1.0, jnp.float32)

    pltpu.sync_copy(half_v, deg_sh.at[pl.ds(s * _NPT, _NPT)])
    plsc.subcore_barrier()

    @pl.loop(0, _DROWS // 16)
    def _chunk(j):
        base = c * (EROWS // 2) + s * _DROWS + j * 16
        pltpu.sync_copy(dst2d.at[pl.ds(base, 16)], idx_v)

        @pl.loop(0, 16)
        def _row(r):
            pltpu.sync_copy(ones_v, deg_sh.at[idx_v.at[r]], add=True)

    plsc.subcore_barrier()

    @pl.when(c == 0)
    def _():
        pltpu.sync_copy(deg_sh.at[pl.ds(s * _NPT, _NPT)],
                        d0_out.at[pl.ds(s * _NPT, _NPT)])

    @pl.when(c == 1)
    def _():
        pltpu.sync_copy(deg_sh.at[pl.ds(s * _NPT, _NPT)],
                        d1_out.at[pl.ds(s * _NPT, _NPT)])


_deg_call = functools.partial(
    pl.kernel,
    out_type=[jax.ShapeDtypeStruct((NP,), jnp.float32),
              jax.ShapeDtypeStruct((NP,), jnp.float32)],
    mesh=_MESH,
    scratch_types=[
        pltpu.VMEM((_NPT,), jnp.float32),
        pltpu.VMEM((128,), jnp.float32),
        pltpu.VMEM((16, 128), jnp.int32),
        pltpu.VMEM_SHARED((NP,), jnp.float32),
    ],
)(_deg_body)


_CHK = 32


def _prop_body(src2d, dst2d, tab_lo, tab_hi, out_lo, out_hi,
               srcb, dstb, buf0, buf1, acc_sh, sem0, sem1):
    c = lax.axis_index("c")
    s = lax.axis_index("s")

    def run(table, out):
        pltpu.sync_copy(table.at[pl.ds(s * _NPT, _NPT)],
                        acc_sh.at[pl.ds(s * _NPT, _NPT)])
        plsc.subcore_barrier()

        @pl.loop(0, TROWS // _CHK)
        def _chunk(j):
            base = s * TROWS + j * _CHK
            pltpu.sync_copy(src2d.at[pl.ds(base, _CHK)], srcb)
            pltpu.sync_copy(dst2d.at[pl.ds(base, _CHK)], dstb)
            pltpu.async_copy(table.at[srcb.at[0]], buf0, sem0)
            pltpu.async_copy(table.at[srcb.at[1]], buf1, sem1)

            @pl.loop(0, _CHK, step=2)
            def _row(r):
                for k, (buf, sem) in enumerate(((buf0, sem0), (buf1, sem1))):
                    idx = r + k
                    pltpu.make_async_copy(table.at[srcb.at[idx]],
                                          buf, sem).wait()
                    pltpu.sync_copy(buf, acc_sh.at[dstb.at[idx]], add=True)

                    @pl.when(idx + 2 < _CHK)
                    def _():
                        pltpu.async_copy(table.at[srcb.at[idx + 2]], buf, sem)

        plsc.subcore_barrier()
        pltpu.sync_copy(acc_sh.at[pl.ds(s * _NPT, _NPT)],
                        out.at[pl.ds(s * _NPT, _NPT)])

    @pl.when(c == 0)
    def _():
        run(tab_lo, out_lo)

    @pl.when(c == 1)
    def _():
        run(tab_hi, out_hi)


_prop_call = functools.partial(
    pl.kernel,
    out_type=[jax.ShapeDtypeStruct((NP, 128), jnp.float32),
              jax.ShapeDtypeStruct((NP, 128), jnp.float32)],
    mesh=_MESH,
    scratch_types=[
        pltpu.VMEM((_CHK, 128), jnp.int32),
        pltpu.VMEM((_CHK, 128), jnp.int32),
        pltpu.VMEM((128, 128), jnp.float32),
        pltpu.VMEM((128, 128), jnp.float32),
        pltpu.VMEM_SHARED((NP, 128), jnp.float32),
        pltpu.SemaphoreType.DMA,
        pltpu.SemaphoreType.DMA,
    ],
)(_prop_body)


def _prep_body(d0_ref, d1_ref, x_ref, w1_ref, zlo_ref, zhi_ref):
    dis = lax.rsqrt(d0_ref[...] + d1_ref[...])
    z = jnp.dot(x_ref[...], w1_ref[...], preferred_element_type=jnp.float32)
    zs = z * dis
    zlo_ref[...] = zs[:, :128]
    zhi_ref[...] = zs[:, 128:]


def _mid_body(d0_ref, d1_ref, alo_ref, ahi_ref, wt_ref, wb_ref, blo_ref,
              bhi_ref, zlo_ref, zhi_ref):
    dis = lax.rsqrt(d0_ref[...] + d1_ref[...])
    h_lo = jax.nn.relu(alo_ref[...] * dis + blo_ref[...])
    h_hi = jax.nn.relu(ahi_ref[...] * dis + bhi_ref[...])
    z2 = (jnp.dot(h_lo, wt_ref[...], preferred_element_type=jnp.float32)
          + jnp.dot(h_hi, wb_ref[...], preferred_element_type=jnp.float32))
    zs2 = z2 * dis
    zlo_ref[...] = zs2[:, :128]
    zhi_ref[...] = zs2[:, 128:]


def _final_body(d0_ref, d1_ref, alo_ref, ahi_ref, bmu_ref, bls_ref,
                mu_ref, ls_ref):
    dis = lax.rsqrt(d0_ref[...] + d1_ref[...])
    mu_ref[...] = alo_ref[...] * dis + bmu_ref[...]
    ls_ref[...] = ahi_ref[...] * dis + bls_ref[...]


_BLK = 1024
_GRID = NP // _BLK

_row_spec = pl.BlockSpec((_BLK, 128), lambda i: (i, 0))
_deg_spec = pl.BlockSpec((_BLK, 1), lambda i: (i, 0))
_bias_spec = pl.BlockSpec((1, 128), lambda i: (0, 0))


def _prep_call(d0, d1, x_pad, w1):
    return pl.pallas_call(
        _prep_body,
        grid=(_GRID,),
        in_specs=[_deg_spec, _deg_spec, _row_spec,
                  pl.BlockSpec((DI, DH), lambda i: (0, 0))],
        out_specs=[_row_spec, _row_spec],
        out_shape=[jax.ShapeDtypeStruct((NP, 128), jnp.float32)] * 2,
    )(d0, d1, x_pad, w1)


def _mid_call(d0, d1, alo, ahi, wt, wb, blo, bhi):
    return pl.pallas_call(
        _mid_body,
        grid=(_GRID,),
        in_specs=[_deg_spec, _deg_spec, _row_spec, _row_spec,
                  pl.BlockSpec((128, DH), lambda i: (0, 0)),
                  pl.BlockSpec((128, DH), lambda i: (0, 0)),
                  _bias_spec, _bias_spec],
        out_specs=[_row_spec, _row_spec],
        out_shape=[jax.ShapeDtypeStruct((NP, 128), jnp.float32)] * 2,
    )(d0, d1, alo, ahi, wt, wb, blo, bhi)


_FBLK = 1000
_frow_spec = pl.BlockSpec((_FBLK, 128), lambda i: (i, 0))
_fdeg_spec = pl.BlockSpec((_FBLK, 1), lambda i: (i, 0))


def _final_call(d0, d1, alo, ahi, bmu, bls):
    return pl.pallas_call(
        _final_body,
        grid=(NN // _FBLK,),
        in_specs=[_fdeg_spec, _fdeg_spec, _frow_spec, _frow_spec,
                  _bias_spec, _bias_spec],
        out_specs=[_frow_spec, _frow_spec],
        out_shape=[jax.ShapeDtypeStruct((NN, 128), jnp.float32)] * 2,
    )(d0, d1, alo, ahi, bmu, bls)


def kernel(x, edge_index, W1, b1, W_mu, b_mu, W_ls, b_ls):
    src = edge_index[0]
    dst = edge_index[1]

    npad = EP - EE
    pad_src = (jnp.arange(npad, dtype=jnp.int32) * 61) % NN
    pad_dst = NN + (jnp.arange(npad, dtype=jnp.int32) % (NP - NN))
    src2d = jnp.concatenate([src, pad_src]).reshape(EROWS, 128)
    dst2d = jnp.concatenate([dst, pad_dst]).reshape(EROWS, 128)

    x_pad = jnp.pad(x, ((0, NP - NN), (0, 0)))

    wt = jnp.concatenate([W_mu[:128], W_ls[:128]], axis=1)
    wb = jnp.concatenate([W_mu[128:], W_ls[128:]], axis=1)
    blo = b1[:128].reshape(1, 128)
    bhi = b1[128:].reshape(1, 128)
    bmu = b_mu.reshape(1, 128)
    bls = b_ls.reshape(1, 128)

    d0, d1 = _deg_call(dst2d)
    d0 = d0.reshape(NP, 1)
    d1 = d1.reshape(NP, 1)

    zs_lo, zs_hi = _prep_call(d0, d1, x_pad, W1)
    acc_lo, acc_hi = _prop_call(src2d, dst2d, zs_lo, zs_hi)
    zs2_lo, zs2_hi = _mid_call(d0, d1, acc_lo, acc_hi, wt, wb, blo, bhi)
    acc2_lo, acc2_hi = _prop_call(src2d, dst2d, zs2_lo, zs2_hi)
    return _final_call(d0, d1, acc2_lo, acc2_hi, bmu, bls)


# speedup vs baseline: 1.2195x; 1.0463x over previous
import functools

import jax
import jax.numpy as jnp
from jax import lax
from jax.experimental import pallas as pl
from jax.experimental.pallas import tpu as pltpu
from jax.experimental.pallas import tpu_sc as plsc

NN = 10000
NP = 10240
EE = 320000
EP = 327680
EROWS = EP // 128
TROWS = EROWS // 16
DI = 128
DH = 256
DO = 128

_MESH = plsc.VectorSubcoreMesh(core_axis_name="c", subcore_axis_name="s")
_NPT = NP // 16


_DROWS = EROWS // 2 // 16


def _deg_body(dst2d, d0_out, d1_out, half_v, ones_v, idx_v, deg_sh):
    c = lax.axis_index("c")
    s = lax.axis_index("s")

    @pl.loop(0, _NPT // 16)
    def _fill(i):
        half_v[pl.ds(i * 16, 16)] = jnp.full((16,), 0.5, jnp.float32)

    @pl.loop(0, 128 // 16)
    def _fill1(i):
        ones_v[pl.ds(i * 16, 16)] = jnp.full((16,), 1.0, jnp.float32)

    pltpu.sync_copy(half_v, deg_sh.at[pl.ds(s * _NPT, _NPT)])
    plsc.subcore_barrier()

    @pl.loop(0, _DROWS // 16)
    def _chunk(j):
        base = c * (EROWS // 2) + s * _DROWS + j * 16
        pltpu.sync_copy(dst2d.at[pl.ds(base, 16)], idx_v)

        @pl.loop(0, 16)
        def _row(r):
            pltpu.sync_copy(ones_v, deg_sh.at[idx_v.at[r]], add=True)

    plsc.subcore_barrier()

    @pl.when(c == 0)
    def _():
        pltpu.sync_copy(deg_sh.at[pl.ds(s * _NPT, _NPT)],
                        d0_out.at[pl.ds(s * _NPT, _NPT)])

    @pl.when(c == 1)
    def _():
        pltpu.sync_copy(deg_sh.at[pl.ds(s * _NPT, _NPT)],
                        d1_out.at[pl.ds(s * _NPT, _NPT)])


_deg_call = functools.partial(
    pl.kernel,
    out_type=[jax.ShapeDtypeStruct((NP,), jnp.float32),
              jax.ShapeDtypeStruct((NP,), jnp.float32)],
    mesh=_MESH,
    scratch_types=[
        pltpu.VMEM((_NPT,), jnp.float32),
        pltpu.VMEM((128,), jnp.float32),
        pltpu.VMEM((16, 128), jnp.int32),
        pltpu.VMEM_SHARED((NP,), jnp.float32),
    ],
)(_deg_body)


_CHK = 32
_NC = TROWS // _CHK


def _prop_body(src2d, dst2d, tab_lo, tab_hi, out_lo, out_hi,
               srcb0, dstb0, srcb1, dstb1, buf0, buf1, acc_sh,
               sem0, sem1, semsi, semdi):
    c = lax.axis_index("c")
    s = lax.axis_index("s")

    def run(table, out):
        sbase = s * TROWS
        pltpu.sync_copy(src2d.at[pl.ds(sbase, _CHK)], srcb0)
        pltpu.sync_copy(dst2d.at[pl.ds(sbase, _CHK)], dstb0)
        pltpu.async_copy(table.at[srcb0.at[0]], buf0, sem0)
        pltpu.async_copy(table.at[srcb0.at[1]], buf1, sem1)
        pltpu.sync_copy(table.at[pl.ds(s * _NPT, _NPT)],
                        acc_sh.at[pl.ds(s * _NPT, _NPT)])
        plsc.subcore_barrier()

        bufs = ((buf0, sem0), (buf1, sem1))
        idxb = ((srcb0, dstb0), (srcb1, dstb1))
        for j in range(_NC):
            cs, cd = idxb[j % 2]
            ns, nd = idxb[(j + 1) % 2]
            nbase = sbase + (j + 1) * _CHK
            if j + 1 < _NC:
                pltpu.async_copy(src2d.at[pl.ds(nbase, _CHK)], ns, semsi)
                pltpu.async_copy(dst2d.at[pl.ds(nbase, _CHK)], nd, semdi)

            @pl.loop(0, _CHK - 2, step=2)
            def _row(r, cs=cs, cd=cd):
                for k, (buf, sem) in enumerate(bufs):
                    idx = r + k
                    pltpu.make_async_copy(table.at[cs.at[idx]],
                                          buf, sem).wait()
                    pltpu.sync_copy(buf, acc_sh.at[cd.at[idx]], add=True)
                    pltpu.async_copy(table.at[cs.at[idx + 2]], buf, sem)

            if j + 1 < _NC:
                pltpu.make_async_copy(src2d.at[pl.ds(nbase, _CHK)],
                                      ns, semsi).wait()
                pltpu.make_async_copy(dst2d.at[pl.ds(nbase, _CHK)],
                                      nd, semdi).wait()
            for idx in (_CHK - 2, _CHK - 1):
                buf, sem = bufs[idx % 2]
                pltpu.make_async_copy(table.at[cs.at[idx]], buf, sem).wait()
                pltpu.sync_copy(buf, acc_sh.at[cd.at[idx]], add=True)
                if j + 1 < _NC:
                    pltpu.async_copy(table.at[ns.at[idx - _CHK + 2]],
                                     buf, sem)

        plsc.subcore_barrier()
        pltpu.sync_copy(acc_sh.at[pl.ds(s * _NPT, _NPT)],
                        out.at[pl.ds(s * _NPT, _NPT)])

    @pl.when(c == 0)
    def _():
        run(tab_lo, out_lo)

    @pl.when(c == 1)
    def _():
        run(tab_hi, out_hi)


_prop_call = functools.partial(
    pl.kernel,
    out_type=[jax.ShapeDtypeStruct((NP, 128), jnp.float32),
              jax.ShapeDtypeStruct((NP, 128), jnp.float32)],
    mesh=_MESH,
    scratch_types=[
        pltpu.VMEM((_CHK, 128), jnp.int32),
        pltpu.VMEM((_CHK, 128), jnp.int32),
        pltpu.VMEM((_CHK, 128), jnp.int32),
        pltpu.VMEM((_CHK, 128), jnp.int32),
        pltpu.VMEM((128, 128), jnp.float32),
        pltpu.VMEM((128, 128), jnp.float32),
        pltpu.VMEM_SHARED((NP, 128), jnp.float32),
        pltpu.SemaphoreType.DMA,
        pltpu.SemaphoreType.DMA,
        pltpu.SemaphoreType.DMA,
        pltpu.SemaphoreType.DMA,
    ],
)(_prop_body)


def _prep_body(d0_ref, d1_ref, x_ref, w1_ref, zlo_ref, zhi_ref):
    dis = lax.rsqrt(d0_ref[...] + d1_ref[...])
    z = jnp.dot(x_ref[...], w1_ref[...], preferred_element_type=jnp.float32)
    zs = z * dis
    zlo_ref[...] = zs[:, :128]
    zhi_ref[...] = zs[:, 128:]


def _mid_body(d0_ref, d1_ref, alo_ref, ahi_ref, wt_ref, wb_ref, blo_ref,
              bhi_ref, zlo_ref, zhi_ref):
    dis = lax.rsqrt(d0_ref[...] + d1_ref[...])
    h_lo = jax.nn.relu(alo_ref[...] * dis + blo_ref[...])
    h_hi = jax.nn.relu(ahi_ref[...] * dis + bhi_ref[...])
    z2 = (jnp.dot(h_lo, wt_ref[...], preferred_element_type=jnp.float32)
          + jnp.dot(h_hi, wb_ref[...], preferred_element_type=jnp.float32))
    zs2 = z2 * dis
    zlo_ref[...] = zs2[:, :128]
    zhi_ref[...] = zs2[:, 128:]


def _final_body(d0_ref, d1_ref, alo_ref, ahi_ref, bmu_ref, bls_ref,
                mu_ref, ls_ref):
    dis = lax.rsqrt(d0_ref[...] + d1_ref[...])
    mu_ref[...] = alo_ref[...] * dis + bmu_ref[...]
    ls_ref[...] = ahi_ref[...] * dis + bls_ref[...]


_BLK = 1024
_GRID = NP // _BLK

_row_spec = pl.BlockSpec((_BLK, 128), lambda i: (i, 0))
_deg_spec = pl.BlockSpec((_BLK, 1), lambda i: (i, 0))
_bias_spec = pl.BlockSpec((1, 128), lambda i: (0, 0))


def _prep_call(d0, d1, x_pad, w1):
    return pl.pallas_call(
        _prep_body,
        grid=(_GRID,),
        in_specs=[_deg_spec, _deg_spec, _row_spec,
                  pl.BlockSpec((DI, DH), lambda i: (0, 0))],
        out_specs=[_row_spec, _row_spec],
        out_shape=[jax.ShapeDtypeStruct((NP, 128), jnp.float32)] * 2,
    )(d0, d1, x_pad, w1)


def _mid_call(d0, d1, alo, ahi, wt, wb, blo, bhi):
    return pl.pallas_call(
        _mid_body,
        grid=(_GRID,),
        in_specs=[_deg_spec, _deg_spec, _row_spec, _row_spec,
                  pl.BlockSpec((128, DH), lambda i: (0, 0)),
                  pl.BlockSpec((128, DH), lambda i: (0, 0)),
                  _bias_spec, _bias_spec],
        out_specs=[_row_spec, _row_spec],
        out_shape=[jax.ShapeDtypeStruct((NP, 128), jnp.float32)] * 2,
    )(d0, d1, alo, ahi, wt, wb, blo, bhi)


_FBLK = 1000
_frow_spec = pl.BlockSpec((_FBLK, 128), lambda i: (i, 0))
_fdeg_spec = pl.BlockSpec((_FBLK, 1), lambda i: (i, 0))


def _final_call(d0, d1, alo, ahi, bmu, bls):
    return pl.pallas_call(
        _final_body,
        grid=(NN // _FBLK,),
        in_specs=[_fdeg_spec, _fdeg_spec, _frow_spec, _frow_spec,
                  _bias_spec, _bias_spec],
        out_specs=[_frow_spec, _frow_spec],
        out_shape=[jax.ShapeDtypeStruct((NN, 128), jnp.float32)] * 2,
    )(d0, d1, alo, ahi, bmu, bls)


def kernel(x, edge_index, W1, b1, W_mu, b_mu, W_ls, b_ls):
    src = edge_index[0]
    dst = edge_index[1]

    npad = EP - EE
    pad_src = (jnp.arange(npad, dtype=jnp.int32) * 61) % NN
    pad_dst = NN + (jnp.arange(npad, dtype=jnp.int32) % (NP - NN))
    src2d = jnp.concatenate([src, pad_src]).reshape(EROWS, 128)
    dst2d = jnp.concatenate([dst, pad_dst]).reshape(EROWS, 128)

    x_pad = jnp.pad(x, ((0, NP - NN), (0, 0)))

    wt = jnp.concatenate([W_mu[:128], W_ls[:128]], axis=1)
    wb = jnp.concatenate([W_mu[128:], W_ls[128:]], axis=1)
    blo = b1[:128].reshape(1, 128)
    bhi = b1[128:].reshape(1, 128)
    bmu = b_mu.reshape(1, 128)
    bls = b_ls.reshape(1, 128)

    d0, d1 = _deg_call(dst2d)
    d0 = d0.reshape(NP, 1)
    d1 = d1.reshape(NP, 1)

    zs_lo, zs_hi = _prep_call(d0, d1, x_pad, W1)
    acc_lo, acc_hi = _prop_call(src2d, dst2d, zs_lo, zs_hi)
    zs2_lo, zs2_hi = _mid_call(d0, d1, acc_lo, acc_hi, wt, wb, blo, bhi)
    acc2_lo, acc2_hi = _prop_call(src2d, dst2d, zs2_lo, zs2_hi)
    return _final_call(d0, d1, acc2_lo, acc2_hi, bmu, bls)


# speedup vs baseline: 1.4915x; 1.2231x over previous
import functools

import jax
import jax.numpy as jnp
from jax import lax
from jax.experimental import pallas as pl
from jax.experimental.pallas import tpu as pltpu
from jax.experimental.pallas import tpu_sc as plsc

NN = 10000
NP = 10240
EE = 320000
EP = 327680
EROWS = EP // 128
TROWS = EROWS // 16
DI = 128
DH = 256
DO = 128

_MESH = plsc.VectorSubcoreMesh(core_axis_name="c", subcore_axis_name="s")
_NPT = NP // 16


_DROWS = EROWS // 2 // 16


def _deg_body(dst2d, d0_out, d1_out, half_v, ones_v, idx_v, deg_sh):
    c = lax.axis_index("c")
    s = lax.axis_index("s")

    @pl.loop(0, _NPT // 16)
    def _fill(i):
        half_v[pl.ds(i * 16, 16)] = jnp.full((16,), 0.5, jnp.float32)

    @pl.loop(0, 128 // 16)
    def _fill1(i):
        ones_v[pl.ds(i * 16, 16)] = jnp.full((16,), 1.0, jnp.float32)

    pltpu.sync_copy(half_v, deg_sh.at[pl.ds(s * _NPT, _NPT)])
    plsc.subcore_barrier()

    @pl.loop(0, _DROWS // 16)
    def _chunk(j):
        base = c * (EROWS // 2) + s * _DROWS + j * 16
        pltpu.sync_copy(dst2d.at[pl.ds(base, 16)], idx_v)

        @pl.loop(0, 16)
        def _row(r):
            pltpu.sync_copy(ones_v, deg_sh.at[idx_v.at[r]], add=True)

    plsc.subcore_barrier()

    @pl.when(c == 0)
    def _():
        pltpu.sync_copy(deg_sh.at[pl.ds(s * _NPT, _NPT)],
                        d0_out.at[pl.ds(s * _NPT, _NPT)])

    @pl.when(c == 1)
    def _():
        pltpu.sync_copy(deg_sh.at[pl.ds(s * _NPT, _NPT)],
                        d1_out.at[pl.ds(s * _NPT, _NPT)])


_deg_call = functools.partial(
    pl.kernel,
    out_type=[jax.ShapeDtypeStruct((NP,), jnp.float32),
              jax.ShapeDtypeStruct((NP,), jnp.float32)],
    mesh=_MESH,
    scratch_types=[
        pltpu.VMEM((_NPT,), jnp.float32),
        pltpu.VMEM((128,), jnp.float32),
        pltpu.VMEM((16, 128), jnp.int32),
        pltpu.VMEM_SHARED((NP,), jnp.float32),
    ],
)(_deg_body)


def _ring(src2d, dst2d, table, acc_sh, s, ebase, chk, nc,
          srcb0, dstb0, srcb1, dstb1, buf0, buf1, sem0, sem1, semsi, semdi):
    pltpu.sync_copy(src2d.at[pl.ds(ebase, chk)], srcb0)
    pltpu.sync_copy(dst2d.at[pl.ds(ebase, chk)], dstb0)
    pltpu.async_copy(table.at[srcb0.at[0]], buf0, sem0)
    pltpu.async_copy(table.at[srcb0.at[1]], buf1, sem1)
    pltpu.sync_copy(table.at[pl.ds(s * _NPT, _NPT)],
                    acc_sh.at[pl.ds(s * _NPT, _NPT)])
    plsc.subcore_barrier()

    bufs = ((buf0, sem0), (buf1, sem1))
    idxb = ((srcb0, dstb0), (srcb1, dstb1))
    for j in range(nc):
        cs, cd = idxb[j % 2]
        ns, nd = idxb[(j + 1) % 2]
        nbase = ebase + (j + 1) * chk
        if j + 1 < nc:
            pltpu.async_copy(src2d.at[pl.ds(nbase, chk)], ns, semsi)
            pltpu.async_copy(dst2d.at[pl.ds(nbase, chk)], nd, semdi)

        @pl.loop(0, chk - 2, step=2)
        def _row(r, cs=cs, cd=cd):
            for k, (buf, sem) in enumerate(bufs):
                idx = r + k
                pltpu.make_async_copy(table.at[cs.at[idx]], buf, sem).wait()
                pltpu.sync_copy(buf, acc_sh.at[cd.at[idx]], add=True)
                pltpu.async_copy(table.at[cs.at[idx + 2]], buf, sem)

        if j + 1 < nc:
            pltpu.make_async_copy(src2d.at[pl.ds(nbase, chk)],
                                  ns, semsi).wait()
            pltpu.make_async_copy(dst2d.at[pl.ds(nbase, chk)],
                                  nd, semdi).wait()
        for idx in (chk - 2, chk - 1):
            buf, sem = bufs[idx % 2]
            pltpu.make_async_copy(table.at[cs.at[idx]], buf, sem).wait()
            pltpu.sync_copy(buf, acc_sh.at[cd.at[idx]], add=True)
            if j + 1 < nc:
                pltpu.async_copy(table.at[ns.at[idx - chk + 2]], buf, sem)

    plsc.subcore_barrier()


_CHK = 32
_NC = TROWS // _CHK


def _prop_body(src2d, dst2d, tab_lo, tab_hi, out_lo, out_hi,
               srcb0, dstb0, srcb1, dstb1, buf0, buf1, acc_sh,
               sem0, sem1, semsi, semdi):
    c = lax.axis_index("c")
    s = lax.axis_index("s")

    def run(table, out):
        _ring(src2d, dst2d, table, acc_sh, s, s * TROWS, _CHK, _NC,
              srcb0, dstb0, srcb1, dstb1, buf0, buf1,
              sem0, sem1, semsi, semdi)
        pltpu.sync_copy(acc_sh.at[pl.ds(s * _NPT, _NPT)],
                        out.at[pl.ds(s * _NPT, _NPT)])

    @pl.when(c == 0)
    def _():
        run(tab_lo, out_lo)

    @pl.when(c == 1)
    def _():
        run(tab_hi, out_hi)


_prop_call = functools.partial(
    pl.kernel,
    out_type=[jax.ShapeDtypeStruct((NP, 128), jnp.float32),
              jax.ShapeDtypeStruct((NP, 128), jnp.float32)],
    mesh=_MESH,
    scratch_types=[
        pltpu.VMEM((_CHK, 128), jnp.int32),
        pltpu.VMEM((_CHK, 128), jnp.int32),
        pltpu.VMEM((_CHK, 128), jnp.int32),
        pltpu.VMEM((_CHK, 128), jnp.int32),
        pltpu.VMEM((128, 128), jnp.float32),
        pltpu.VMEM((128, 128), jnp.float32),
        pltpu.VMEM_SHARED((NP, 128), jnp.float32),
        pltpu.SemaphoreType.DMA,
        pltpu.SemaphoreType.DMA,
        pltpu.SemaphoreType.DMA,
        pltpu.SemaphoreType.DMA,
    ],
)(_prop_body)


_CHK1 = 16
_T1 = (EROWS // 2) // 16
_NC1 = _T1 // _CHK1


def _prop1_body(src2d, dst2d, tab, out0, out1,
                srcb0, dstb0, srcb1, dstb1, buf0, buf1, acc_sh,
                sem0, sem1, semsi, semdi):
    c = lax.axis_index("c")
    s = lax.axis_index("s")

    ebase = c * (EROWS // 2) + s * _T1
    _ring(src2d, dst2d, tab, acc_sh, s, ebase, _CHK1, _NC1,
          srcb0, dstb0, srcb1, dstb1, buf0, buf1,
          sem0, sem1, semsi, semdi)

    @pl.when(c == 0)
    def _():
        pltpu.sync_copy(acc_sh.at[pl.ds(s * _NPT, _NPT)],
                        out0.at[pl.ds(s * _NPT, _NPT)])

    @pl.when(c == 1)
    def _():
        pltpu.sync_copy(acc_sh.at[pl.ds(s * _NPT, _NPT)],
                        out1.at[pl.ds(s * _NPT, _NPT)])


_prop1_call = functools.partial(
    pl.kernel,
    out_type=[jax.ShapeDtypeStruct((NP, 128), jnp.float32),
              jax.ShapeDtypeStruct((NP, 128), jnp.float32)],
    mesh=_MESH,
    scratch_types=[
        pltpu.VMEM((_CHK1, 128), jnp.int32),
        pltpu.VMEM((_CHK1, 128), jnp.int32),
        pltpu.VMEM((_CHK1, 128), jnp.int32),
        pltpu.VMEM((_CHK1, 128), jnp.int32),
        pltpu.VMEM((128, 128), jnp.float32),
        pltpu.VMEM((128, 128), jnp.float32),
        pltpu.VMEM_SHARED((NP, 128), jnp.float32),
        pltpu.SemaphoreType.DMA,
        pltpu.SemaphoreType.DMA,
        pltpu.SemaphoreType.DMA,
        pltpu.SemaphoreType.DMA,
    ],
)(_prop1_body)


def _prep_body(d0_ref, d1_ref, x_ref, zs_ref):
    dis = lax.rsqrt(d0_ref[...] + d1_ref[...])
    zs_ref[...] = x_ref[...] * dis


def _mid_body(d0_ref, d1_ref, a0_ref, a1_ref, zs1_ref, w1_ref, b1_ref,
              zlo_ref, zhi_ref):
    dis = lax.rsqrt(d0_ref[...] + d1_ref[...])
    p1 = a0_ref[...] + a1_ref[...] - zs1_ref[...]
    z = jnp.dot(p1, w1_ref[...], preferred_element_type=jnp.float32)
    h = jax.nn.relu(z * dis + b1_ref[...])
    zs2 = h * dis
    zlo_ref[...] = zs2[:, :128]
    zhi_ref[...] = zs2[:, 128:]


def _final_body(d0_ref, d1_ref, alo_ref, ahi_ref, w2_ref, bmu_ref, bls_ref,
                mu_ref, ls_ref):
    dis = lax.rsqrt(d0_ref[...] + d1_ref[...])
    p2 = jnp.concatenate([alo_ref[...], ahi_ref[...]], axis=1)
    z = jnp.dot(p2, w2_ref[...], preferred_element_type=jnp.float32)
    mu_ref[...] = z[:, :128] * dis + bmu_ref[...]
    ls_ref[...] = z[:, 128:] * dis + bls_ref[...]


_BLK = 1024
_GRID = NP // _BLK

_row_spec = pl.BlockSpec((_BLK, 128), lambda i: (i, 0))
_deg_spec = pl.BlockSpec((_BLK, 1), lambda i: (i, 0))
_bias_spec = pl.BlockSpec((1, 128), lambda i: (0, 0))


def _prep_call(d0, d1, x_pad):
    return pl.pallas_call(
        _prep_body,
        grid=(_GRID,),
        in_specs=[_deg_spec, _deg_spec, _row_spec],
        out_specs=_row_spec,
        out_shape=jax.ShapeDtypeStruct((NP, 128), jnp.float32),
    )(d0, d1, x_pad)


def _mid_call(d0, d1, a0, a1, zs1, w1, b1r):
    return pl.pallas_call(
        _mid_body,
        grid=(_GRID,),
        in_specs=[_deg_spec, _deg_spec, _row_spec, _row_spec, _row_spec,
                  pl.BlockSpec((DI, DH), lambda i: (0, 0)),
                  pl.BlockSpec((1, DH), lambda i: (0, 0))],
        out_specs=[_row_spec, _row_spec],
        out_shape=[jax.ShapeDtypeStruct((NP, 128), jnp.float32)] * 2,
    )(d0, d1, a0, a1, zs1, w1, b1r)


_FBLK = 1000
_frow_spec = pl.BlockSpec((_FBLK, 128), lambda i: (i, 0))
_fdeg_spec = pl.BlockSpec((_FBLK, 1), lambda i: (i, 0))


def _final_call(d0, d1, alo, ahi, w2, bmu, bls):
    return pl.pallas_call(
        _final_body,
        grid=(NN // _FBLK,),
        in_specs=[_fdeg_spec, _fdeg_spec, _frow_spec, _frow_spec,
                  pl.BlockSpec((DH, DH), lambda i: (0, 0)),
                  _bias_spec, _bias_spec],
        out_specs=[_frow_spec, _frow_spec],
        out_shape=[jax.ShapeDtypeStruct((NN, 128), jnp.float32)] * 2,
    )(d0, d1, alo, ahi, w2, bmu, bls)


def kernel(x, edge_index, W1, b1, W_mu, b_mu, W_ls, b_ls):
    src = edge_index[0]
    dst = edge_index[1]

    npad = EP - EE
    pad_src = (jnp.arange(npad, dtype=jnp.int32) * 61) % NN
    pad_dst = NN + (jnp.arange(npad, dtype=jnp.int32) % (NP - NN))
    src2d = jnp.concatenate([src, pad_src]).reshape(EROWS, 128)
    dst2d = jnp.concatenate([dst, pad_dst]).reshape(EROWS, 128)

    x_pad = jnp.pad(x, ((0, NP - NN), (0, 0)))

    w2 = jnp.concatenate([W_mu, W_ls], axis=1)
    b1r = b1.reshape(1, DH)
    bmu = b_mu.reshape(1, 128)
    bls = b_ls.reshape(1, 128)

    d0, d1 = _deg_call(dst2d)
    d0 = d0.reshape(NP, 1)
    d1 = d1.reshape(NP, 1)

    zs1 = _prep_call(d0, d1, x_pad)
    a0, a1 = _prop1_call(src2d, dst2d, zs1)
    zs2_lo, zs2_hi = _mid_call(d0, d1, a0, a1, zs1, W1, b1r)
    acc2_lo, acc2_hi = _prop_call(src2d, dst2d, zs2_lo, zs2_hi)
    return _final_call(d0, d1, acc2_lo, acc2_hi, w2, bmu, bls)


# speedup vs baseline: 1.5009x; 1.0063x over previous
import functools

import jax
import jax.numpy as jnp
from jax import lax
from jax.experimental import pallas as pl
from jax.experimental.pallas import tpu as pltpu
from jax.experimental.pallas import tpu_sc as plsc

NN = 10000
NP = 10240
EE = 320000
EP = 327680
EROWS = EP // 128
TROWS = EROWS // 16
DI = 128
DH = 256
DO = 128

_MESH = plsc.VectorSubcoreMesh(core_axis_name="c", subcore_axis_name="s")
_NPT = NP // 16


_DROWS = EROWS // 2 // 16


def _deg_body(dst2d, d0_out, d1_out, half_v, ones_v, idx_v, deg_sh):
    c = lax.axis_index("c")
    s = lax.axis_index("s")

    @pl.loop(0, _NPT // 16)
    def _fill(i):
        half_v[pl.ds(i * 16, 16)] = jnp.full((16,), 0.5, jnp.float32)

    @pl.loop(0, 128 // 16)
    def _fill1(i):
        ones_v[pl.ds(i * 16, 16)] = jnp.full((16,), 1.0, jnp.float32)

    pltpu.sync_copy(half_v, deg_sh.at[pl.ds(s * _NPT, _NPT)])
    plsc.subcore_barrier()

    @pl.loop(0, _DROWS // 16)
    def _chunk(j):
        base = c * (EROWS // 2) + s * _DROWS + j * 16
        pltpu.sync_copy(dst2d.at[pl.ds(base, 16)], idx_v)

        @pl.loop(0, 16)
        def _row(r):
            pltpu.sync_copy(ones_v, deg_sh.at[idx_v.at[r]], add=True)

    plsc.subcore_barrier()

    @pl.when(c == 0)
    def _():
        pltpu.sync_copy(deg_sh.at[pl.ds(s * _NPT, _NPT)],
                        d0_out.at[pl.ds(s * _NPT, _NPT)])

    @pl.when(c == 1)
    def _():
        pltpu.sync_copy(deg_sh.at[pl.ds(s * _NPT, _NPT)],
                        d1_out.at[pl.ds(s * _NPT, _NPT)])


_deg_call = functools.partial(
    pl.kernel,
    out_type=[jax.ShapeDtypeStruct((NP,), jnp.float32),
              jax.ShapeDtypeStruct((NP,), jnp.float32)],
    mesh=_MESH,
    scratch_types=[
        pltpu.VMEM((_NPT,), jnp.float32),
        pltpu.VMEM((128,), jnp.float32),
        pltpu.VMEM((16, 128), jnp.int32),
        pltpu.VMEM_SHARED((NP,), jnp.float32),
    ],
)(_deg_body)


def _ring(src2d, dst2d, table, acc_sh, s, ebase, chk, nc,
          srcb0, dstb0, srcb1, dstb1, buf0, buf1, sem0, sem1, semsi, semdi):
    pltpu.sync_copy(src2d.at[pl.ds(ebase, chk)], srcb0)
    pltpu.sync_copy(dst2d.at[pl.ds(ebase, chk)], dstb0)
    pltpu.async_copy(table.at[srcb0.at[0]], buf0, sem0)
    pltpu.async_copy(table.at[srcb0.at[1]], buf1, sem1)
    pltpu.sync_copy(table.at[pl.ds(s * _NPT, _NPT)],
                    acc_sh.at[pl.ds(s * _NPT, _NPT)])
    plsc.subcore_barrier()

    bufs = ((buf0, sem0), (buf1, sem1))
    idxb = ((srcb0, dstb0), (srcb1, dstb1))
    for j in range(nc):
        cs, cd = idxb[j % 2]
        ns, nd = idxb[(j + 1) % 2]
        nbase = ebase + (j + 1) * chk
        if j + 1 < nc:
            pltpu.async_copy(src2d.at[pl.ds(nbase, chk)], ns, semsi)
            pltpu.async_copy(dst2d.at[pl.ds(nbase, chk)], nd, semdi)

        @pl.loop(0, chk - 2, step=2)
        def _row(r, cs=cs, cd=cd):
            for k, (buf, sem) in enumerate(bufs):
                idx = r + k
                pltpu.make_async_copy(table.at[cs.at[idx]], buf, sem).wait()
                pltpu.sync_copy(buf, acc_sh.at[cd.at[idx]], add=True)
                pltpu.async_copy(table.at[cs.at[idx + 2]], buf, sem)

        if j + 1 < nc:
            pltpu.make_async_copy(src2d.at[pl.ds(nbase, chk)],
                                  ns, semsi).wait()
            pltpu.make_async_copy(dst2d.at[pl.ds(nbase, chk)],
                                  nd, semdi).wait()
        for idx in (chk - 2, chk - 1):
            buf, sem = bufs[idx % 2]
            pltpu.make_async_copy(table.at[cs.at[idx]], buf, sem).wait()
            pltpu.sync_copy(buf, acc_sh.at[cd.at[idx]], add=True)
            if j + 1 < nc:
                pltpu.async_copy(table.at[ns.at[idx - chk + 2]], buf, sem)

    plsc.subcore_barrier()


_CHK = 32
_NC = TROWS // _CHK


def _prop_body(src2d, dst2d, tab_lo, tab_hi, out_lo, out_hi,
               srcb0, dstb0, srcb1, dstb1, buf0, buf1, acc_sh,
               sem0, sem1, semsi, semdi):
    c = lax.axis_index("c")
    s = lax.axis_index("s")

    def run(table, out):
        _ring(src2d, dst2d, table, acc_sh, s, s * TROWS, _CHK, _NC,
              srcb0, dstb0, srcb1, dstb1, buf0, buf1,
              sem0, sem1, semsi, semdi)
        pltpu.sync_copy(acc_sh.at[pl.ds(s * _NPT, _NPT)],
                        out.at[pl.ds(s * _NPT, _NPT)])

    @pl.when(c == 0)
    def _():
        run(tab_lo, out_lo)

    @pl.when(c == 1)
    def _():
        run(tab_hi, out_hi)


_prop_call = functools.partial(
    pl.kernel,
    out_type=[jax.ShapeDtypeStruct((NP, 128), jnp.float32),
              jax.ShapeDtypeStruct((NP, 128), jnp.float32)],
    mesh=_MESH,
    scratch_types=[
        pltpu.VMEM((_CHK, 128), jnp.int32),
        pltpu.VMEM((_CHK, 128), jnp.int32),
        pltpu.VMEM((_CHK, 128), jnp.int32),
        pltpu.VMEM((_CHK, 128), jnp.int32),
        pltpu.VMEM((128, 128), jnp.float32),
        pltpu.VMEM((128, 128), jnp.float32),
        pltpu.VMEM_SHARED((NP, 128), jnp.float32),
        pltpu.SemaphoreType.DMA,
        pltpu.SemaphoreType.DMA,
        pltpu.SemaphoreType.DMA,
        pltpu.SemaphoreType.DMA,
    ],
)(_prop_body)


_CHK1 = 16
_T1 = (EROWS // 2) // 16
_NC1 = _T1 // _CHK1


def _prop1_body(src2d, dst2d, tab, out0, out1,
                srcb0, dstb0, srcb1, dstb1, buf0, buf1, acc_sh,
                sem0, sem1, semsi, semdi):
    c = lax.axis_index("c")
    s = lax.axis_index("s")

    ebase = c * (EROWS // 2) + s * _T1
    _ring(src2d, dst2d, tab, acc_sh, s, ebase, _CHK1, _NC1,
          srcb0, dstb0, srcb1, dstb1, buf0, buf1,
          sem0, sem1, semsi, semdi)

    @pl.when(c == 0)
    def _():
        pltpu.sync_copy(acc_sh.at[pl.ds(s * _NPT, _NPT)],
                        out0.at[pl.ds(s * _NPT, _NPT)])

    @pl.when(c == 1)
    def _():
        pltpu.sync_copy(acc_sh.at[pl.ds(s * _NPT, _NPT)],
                        out1.at[pl.ds(s * _NPT, _NPT)])


_prop1_call = functools.partial(
    pl.kernel,
    out_type=[jax.ShapeDtypeStruct((NP, 128), jnp.float32),
              jax.ShapeDtypeStruct((NP, 128), jnp.float32)],
    mesh=_MESH,
    scratch_types=[
        pltpu.VMEM((_CHK1, 128), jnp.int32),
        pltpu.VMEM((_CHK1, 128), jnp.int32),
        pltpu.VMEM((_CHK1, 128), jnp.int32),
        pltpu.VMEM((_CHK1, 128), jnp.int32),
        pltpu.VMEM((128, 128), jnp.float32),
        pltpu.VMEM((128, 128), jnp.float32),
        pltpu.VMEM_SHARED((NP, 128), jnp.float32),
        pltpu.SemaphoreType.DMA,
        pltpu.SemaphoreType.DMA,
        pltpu.SemaphoreType.DMA,
        pltpu.SemaphoreType.DMA,
    ],
)(_prop1_body)


def _prep_body(d0_ref, d1_ref, x_ref, zs_ref):
    dis = lax.rsqrt(d0_ref[...] + d1_ref[...])
    zs_ref[...] = x_ref[...] * dis


def _mid_body(d0_ref, d1_ref, a0_ref, a1_ref, zs1_ref, w1_ref, b1_ref,
              zlo_ref, zhi_ref):
    dis = lax.rsqrt(d0_ref[...] + d1_ref[...])
    p1 = a0_ref[...] + a1_ref[...] - zs1_ref[...]
    z = jnp.dot(p1, w1_ref[...], preferred_element_type=jnp.float32)
    h = jax.nn.relu(z * dis + b1_ref[...])
    zs2 = h * dis
    zlo_ref[...] = zs2[:, :128]
    zhi_ref[...] = zs2[:, 128:]


def _final_body(d0_ref, d1_ref, alo_ref, ahi_ref, w2_ref, bmu_ref, bls_ref,
                mu_ref, ls_ref):
    dis = lax.rsqrt(d0_ref[...] + d1_ref[...])
    p2 = jnp.concatenate([alo_ref[...], ahi_ref[...]], axis=1)
    z = jnp.dot(p2, w2_ref[...], preferred_element_type=jnp.float32)
    mu_ref[...] = z[:, :128] * dis + bmu_ref[...]
    ls_ref[...] = z[:, 128:] * dis + bls_ref[...]


_BLK = 1024
_GRID = NP // _BLK

_FBLK = 1000
_frow_spec = pl.BlockSpec((_FBLK, 128), lambda i: (i, 0))
_fdeg_spec = pl.BlockSpec((_FBLK, 1), lambda i: (i, 0))

_row_spec = pl.BlockSpec((_BLK, 128), lambda i: (i, 0))
_deg_spec = pl.BlockSpec((_BLK, 1), lambda i: (i, 0))
_bias_spec = pl.BlockSpec((1, 128), lambda i: (0, 0))


def _prep_call(d0, d1, x):
    return pl.pallas_call(
        _prep_body,
        grid=(NN // _FBLK,),
        in_specs=[_fdeg_spec, _fdeg_spec, _frow_spec],
        out_specs=_frow_spec,
        out_shape=jax.ShapeDtypeStruct((NP, 128), jnp.float32),
    )(d0, d1, x)


def _mid_call(d0, d1, a0, a1, zs1, w1, b1r):
    return pl.pallas_call(
        _mid_body,
        grid=(_GRID,),
        in_specs=[_deg_spec, _deg_spec, _row_spec, _row_spec, _row_spec,
                  pl.BlockSpec((DI, DH), lambda i: (0, 0)),
                  pl.BlockSpec((1, DH), lambda i: (0, 0))],
        out_specs=[_row_spec, _row_spec],
        out_shape=[jax.ShapeDtypeStruct((NP, 128), jnp.float32)] * 2,
    )(d0, d1, a0, a1, zs1, w1, b1r)


def _final_call(d0, d1, alo, ahi, w2, bmu, bls):
    return pl.pallas_call(
        _final_body,
        grid=(NN // _FBLK,),
        in_specs=[_fdeg_spec, _fdeg_spec, _frow_spec, _frow_spec,
                  pl.BlockSpec((DH, DH), lambda i: (0, 0)),
                  _bias_spec, _bias_spec],
        out_specs=[_frow_spec, _frow_spec],
        out_shape=[jax.ShapeDtypeStruct((NN, 128), jnp.float32)] * 2,
    )(d0, d1, alo, ahi, w2, bmu, bls)


def kernel(x, edge_index, W1, b1, W_mu, b_mu, W_ls, b_ls):
    src = edge_index[0]
    dst = edge_index[1]

    npad = EP - EE
    pad_src = (jnp.arange(npad, dtype=jnp.int32) * 61) % NN
    pad_dst = NN + (jnp.arange(npad, dtype=jnp.int32) % (NP - NN))
    src2d = jnp.concatenate([src, pad_src]).reshape(EROWS, 128)
    dst2d = jnp.concatenate([dst, pad_dst]).reshape(EROWS, 128)

    w2 = jnp.concatenate([W_mu, W_ls], axis=1)
    b1r = b1.reshape(1, DH)
    bmu = b_mu.reshape(1, 128)
    bls = b_ls.reshape(1, 128)

    d0, d1 = _deg_call(dst2d)
    d0 = d0.reshape(NP, 1)
    d1 = d1.reshape(NP, 1)

    zs1 = _prep_call(d0, d1, x)
    a0, a1 = _prop1_call(src2d, dst2d, zs1)
    zs2_lo, zs2_hi = _mid_call(d0, d1, a0, a1, zs1, W1, b1r)
    acc2_lo, acc2_hi = _prop_call(src2d, dst2d, zs2_lo, zs2_hi)
    return _final_call(d0, d1, acc2_lo, acc2_hi, w2, bmu, bls)
